# Initial kernel scaffold; baseline (speedup 1.0000x reference)
import jax
import jax.numpy as jnp
from jax.experimental import pallas as pl


def kernel(hours, days, deltas_hours, velocities, hour_table, circ_w1, circ_b1, circ_w2, circ_b2, day_table, wk_w, wk_b, scale_table, mag_w1, mag_b1, mag_w2, mag_b2, vel_w1, vel_b1, vel_w2, vel_b2, comp_w1, comp_b1, comp_w2, comp_b2):
    raise NotImplementedError("write your pallas kernel here")


# speedup vs baseline: 3.0973x; 3.0973x over previous
import math

import jax
import jax.numpy as jnp
from jax.experimental import pallas as pl
from jax.experimental.pallas import tpu as pltpu

_NB = 4096


def _fused_weights(hour_table, circ_w1, circ_b1, circ_w2, circ_b2, day_table,
                   wk_w, wk_b, scale_table, mag_w2, mag_b2, vel_w2, vel_b2,
                   comp_w1, comp_b1):
    hh = jax.lax.broadcasted_iota(jnp.int32, (24, 1), 0).astype(jnp.float32)
    ang = (2.0 * math.pi / 24.0) * hh
    phase = jnp.concatenate([jnp.sin(ang), jnp.cos(ang)], axis=1)
    cont = jnp.maximum(phase @ circ_w1 + circ_b1, 0.0) @ circ_w2 + circ_b2
    t24 = jnp.concatenate([hour_table, cont], axis=1) @ comp_w1[0:48]
    is_wk = (jax.lax.broadcasted_iota(jnp.int32, (7, 1), 0) >= 5).astype(jnp.float32)
    t7 = jnp.concatenate([day_table, is_wk @ wk_w + wk_b], axis=1) @ comp_w1[48:64]
    t3 = scale_table @ comp_w1[64:69]
    mw = mag_w2 @ comp_w1[69:74]
    dw = comp_w1[74:76]
    vw = vel_w2 @ comp_w1[76:84]
    pad = jnp.zeros((128 - 49, 128), jnp.float32)
    wf = jnp.concatenate([t24, t7, t3, mw, dw, vw, pad], axis=0)
    bf = comp_b1 + mag_b2 @ comp_w1[69:74] + vel_b2 @ comp_w1[76:84]
    return wf, bf


def _kern(hf_ref, df_ref, dt_ref, vel_ref, hour_table, circ_w1, circ_b1,
          circ_w2, circ_b2, day_table, wk_w, wk_b, scale_table, mag_w1,
          mag_b1, mag_w2, mag_b2, vel_w1, vel_b1, vel_w2, vel_b2, comp_w1,
          comp_b1, comp_w2, comp_b2, out_ref):
    wf, bf = _fused_weights(hour_table[...], circ_w1[...], circ_b1[...],
                            circ_w2[...], circ_b2[...], day_table[...],
                            wk_w[...], wk_b[...], scale_table[...],
                            mag_w2[...], mag_b2[...], vel_w2[...],
                            vel_b2[...], comp_w1[...], comp_b1[...])
    z = jnp.zeros((1, 34), jnp.float32)
    z2 = jnp.zeros((1, 2), jnp.float32)
    zt = jnp.zeros((1, 79), jnp.float32)
    wrow = jnp.concatenate([z, mag_w1[...], z2, vel_w1[...], zt], axis=1)
    brow = jnp.concatenate([z, mag_b1[...], z2, vel_b1[...], zt], axis=1)

    hf = hf_ref[...]
    df = df_ref[...]
    dt = dt_ref[...]
    v = vel_ref[...]
    dc = jnp.clip(dt, 0.0, 24.0)
    mins = dc * 60.0
    sf = jnp.where(mins < 5.0, 0, jnp.where(mins < 60.0, 1, 2)).astype(jnp.int32)
    lm = jnp.log1p(dc * (1.0 / 24.0))
    m60 = mins - 60.0 * jnp.floor(mins * (1.0 / 60.0))
    a = m60 * (2.0 * math.pi / 60.0)
    s = jnp.sin(a)
    c = jnp.cos(a)

    lane = jax.lax.broadcasted_iota(jnp.int32, (hf.shape[0], 128), 1)
    oh = ((lane == hf) | (lane == df + 24) | (lane == sf + 31)).astype(jnp.float32)
    cont_in = jnp.where(lane >= 41, v, lm)
    g = jnp.maximum(cont_in * wrow + brow, 0.0)
    sc = jnp.where(lane == 39, s, 0.0) + jnp.where(lane == 40, c, 0.0)
    f = oh + g + sc

    h1 = jnp.maximum(f @ wf + bf, 0.0)
    out_ref[...] = h1 @ comp_w2[...] + comp_b2[...]


def kernel(hours, days, deltas_hours, velocities, hour_table, circ_w1,
           circ_b1, circ_w2, circ_b2, day_table, wk_w, wk_b, scale_table,
           mag_w1, mag_b1, mag_w2, mag_b2, vel_w1, vel_b1, vel_w2, vel_b2,
           comp_w1, comp_b1, comp_w2, comp_b2):
    B, S = hours.shape
    n = B * S
    nb = _NB
    hf = hours.reshape(n, 1)
    df = days.reshape(n, 1)
    dt = deltas_hours.reshape(n, 1)
    v = velocities.reshape(n, 1)

    def row2(x):
        return x.reshape(1, -1)

    tok_spec = pl.BlockSpec((nb, 1), lambda i: (i, 0))
    full = lambda a: pl.BlockSpec(a.shape, lambda i: tuple(0 for _ in a.shape))
    weights = (hour_table, circ_w1, row2(circ_b1), circ_w2, row2(circ_b2),
               day_table, wk_w, row2(wk_b), scale_table, mag_w1, row2(mag_b1),
               mag_w2, row2(mag_b2), vel_w1, row2(vel_b1), vel_w2,
               row2(vel_b2), comp_w1, row2(comp_b1), comp_w2, row2(comp_b2))
    out = pl.pallas_call(
        _kern,
        grid=(n // nb,),
        in_specs=[tok_spec, tok_spec, tok_spec, tok_spec] + [full(w) for w in weights],
        out_specs=pl.BlockSpec((nb, 64), lambda i: (i, 0)),
        out_shape=jax.ShapeDtypeStruct((n, 64), jnp.float32),
        compiler_params=pltpu.CompilerParams(
            dimension_semantics=("arbitrary",)),
    )(hf, df, dt, v, *weights)
    return out.reshape(B, S, 64)


# speedup vs baseline: 12.0420x; 3.8880x over previous
import math

import jax
import jax.numpy as jnp
from jax.experimental import pallas as pl
from jax.experimental.pallas import tpu as pltpu

_NT = 8192


def _scalars_kern(dt_ref, s_ref, c_ref, lm_ref, sf_ref):
    dt = dt_ref[...]
    dc = jnp.clip(dt, 0.0, 24.0)
    mins = dc * 60.0
    sf_ref[...] = jnp.where(mins < 5.0, 0,
                            jnp.where(mins < 60.0, 1, 2)).astype(jnp.int32)
    lm_ref[...] = jnp.log1p(dc * (1.0 / 24.0))
    m60 = mins - 60.0 * jnp.floor(mins * (1.0 / 60.0))
    a = m60 * (2.0 * math.pi / 60.0)
    ac = jnp.concatenate([a, a + 0.5 * math.pi], axis=0)
    sc = jnp.sin(ac)
    r = a.shape[0]
    s_ref[...] = sc[:r]
    c_ref[...] = sc[r:]


def _fused_weights(hour_table, circ_w1, circ_b1, circ_w2, circ_b2, day_table,
                   wk_w, wk_b, scale_table, mag_w2, mag_b2, vel_w2, vel_b2,
                   comp_w1, comp_b1):
    hh = jax.lax.broadcasted_iota(jnp.int32, (24, 1), 0).astype(jnp.float32)
    ang = (2.0 * math.pi / 24.0) * hh
    phase = jnp.concatenate([jnp.sin(ang), jnp.cos(ang)], axis=1)
    cont = jnp.maximum(phase @ circ_w1 + circ_b1, 0.0) @ circ_w2 + circ_b2
    t24 = jnp.concatenate([hour_table, cont], axis=1) @ comp_w1[0:48]
    is_wk = (jax.lax.broadcasted_iota(jnp.int32, (7, 1), 0) >= 5).astype(jnp.float32)
    t7 = jnp.concatenate([day_table, is_wk @ wk_w + wk_b], axis=1) @ comp_w1[48:64]
    t3 = scale_table @ comp_w1[64:69]
    mw = mag_w2 @ comp_w1[69:74]
    dw = comp_w1[74:76]
    vw = vel_w2 @ comp_w1[76:84]
    wf = jnp.concatenate([t24, t7, t3, mw, dw, vw], axis=0)
    bf = comp_b1 + mag_b2 @ comp_w1[69:74] + vel_b2 @ comp_w1[76:84]
    return wf, bf


def _main_kern(hf_ref, df_ref, v_ref, s_ref, c_ref, lm_ref, sf_ref,
               hour_table, circ_w1, circ_b1, circ_w2, circ_b2, day_table,
               wk_w, wk_b, scale_table, mag_w1, mag_b1, mag_w2, mag_b2,
               vel_w1, vel_b1, vel_w2, vel_b2, comp_w1, comp_b1, comp_w2,
               comp_b2, out_ref):
    wf, bf = _fused_weights(hour_table[...], circ_w1[...], circ_b1[...],
                            circ_w2[...], circ_b2[...], day_table[...],
                            wk_w[...], wk_b[...], scale_table[...],
                            mag_w2[...], mag_b2[...], vel_w2[...],
                            vel_b2[...], comp_w1[...], comp_b1[...])
    hf = hf_ref[...]
    df = df_ref[...]
    sf = sf_ref[...]
    v = v_ref[...]
    s = s_ref[...]
    c = c_ref[...]
    lm = lm_ref[...]
    nt = hf.shape[1]

    oh24 = (jax.lax.broadcasted_iota(jnp.int32, (24, nt), 0) == hf
            ).astype(jnp.float32)
    oh7 = (jax.lax.broadcasted_iota(jnp.int32, (7, nt), 0) == df
           ).astype(jnp.float32)
    oh3 = (jax.lax.broadcasted_iota(jnp.int32, (3, nt), 0) == sf
           ).astype(jnp.float32)
    hm = jnp.maximum(lm * mag_w1[...].T + mag_b1[...].T, 0.0)
    hv = jnp.maximum(v * vel_w1[...].T + vel_b1[...].T, 0.0)
    ft = jnp.concatenate([oh24, oh7, oh3, hm, s, c, hv], axis=0)

    h1 = jax.lax.dot_general(ft, wf, (((0,), (0,)), ((), ())),
                             preferred_element_type=jnp.float32)
    h1 = jnp.maximum(h1 + bf, 0.0)
    out_ref[...] = h1 @ comp_w2[...] + comp_b2[...]


def kernel(hours, days, deltas_hours, velocities, hour_table, circ_w1,
           circ_b1, circ_w2, circ_b2, day_table, wk_w, wk_b, scale_table,
           mag_w1, mag_b1, mag_w2, mag_b2, vel_w1, vel_b1, vel_w2, vel_b2,
           comp_w1, comp_b1, comp_w2, comp_b2):
    B, S = hours.shape
    n = B * S
    nt = _NT
    dt_p = deltas_hours.reshape(n // 128, 128)

    s_p, c_p, lm_p, sf_p = pl.pallas_call(
        _scalars_kern,
        out_shape=[jax.ShapeDtypeStruct(dt_p.shape, jnp.float32)] * 3
        + [jax.ShapeDtypeStruct(dt_p.shape, jnp.int32)],
    )(dt_p)

    def row(x):
        return x.reshape(1, n)

    def row2(x):
        return x.reshape(1, -1)

    tok_spec = pl.BlockSpec((1, nt), lambda i: (0, i))
    full = lambda a: pl.BlockSpec(a.shape, lambda i: tuple(0 for _ in a.shape))
    weights = (hour_table, circ_w1, row2(circ_b1), circ_w2, row2(circ_b2),
               day_table, wk_w, row2(wk_b), scale_table, mag_w1, row2(mag_b1),
               mag_w2, row2(mag_b2), vel_w1, row2(vel_b1), vel_w2,
               row2(vel_b2), comp_w1, row2(comp_b1), comp_w2, row2(comp_b2))
    toks = (row(hours), row(days), row(velocities), row(s_p), row(c_p),
            row(lm_p), row(sf_p))
    out = pl.pallas_call(
        _main_kern,
        grid=(n // nt,),
        in_specs=[tok_spec] * 7 + [full(w) for w in weights],
        out_specs=pl.BlockSpec((nt, 64), lambda i: (i, 0)),
        out_shape=jax.ShapeDtypeStruct((n, 64), jnp.float32),
        compiler_params=pltpu.CompilerParams(
            dimension_semantics=("arbitrary",)),
    )(*toks, *weights)
    return out.reshape(B, S, 64)


# speedup vs baseline: 18.8546x; 1.5657x over previous
import math

import jax
import jax.numpy as jnp
from jax.experimental import pallas as pl
from jax.experimental.pallas import tpu as pltpu

_BB = 128
_SP = 56
_NT = _BB * _SP


def _fused_weights(hour_table, circ_w1, circ_b1, circ_w2, circ_b2, day_table,
                   wk_w, wk_b, scale_table, mag_w2, mag_b2, vel_w2, vel_b2,
                   comp_w1, comp_b1):
    hh = jax.lax.broadcasted_iota(jnp.int32, (24, 1), 0).astype(jnp.float32)
    ang = (2.0 * math.pi / 24.0) * hh
    phase = jnp.concatenate([jnp.sin(ang), jnp.cos(ang)], axis=1)
    cont = jnp.maximum(phase @ circ_w1 + circ_b1, 0.0) @ circ_w2 + circ_b2
    t24 = jnp.concatenate([hour_table, cont], axis=1) @ comp_w1[0:48]
    is_wk = (jax.lax.broadcasted_iota(jnp.int32, (7, 1), 0) >= 5).astype(jnp.float32)
    t7 = jnp.concatenate([day_table, is_wk @ wk_w + wk_b], axis=1) @ comp_w1[48:64]
    t3 = scale_table @ comp_w1[64:69]
    mw = mag_w2 @ comp_w1[69:74]
    dw = comp_w1[74:76]
    vw = vel_w2 @ comp_w1[76:84]
    bf = comp_b1 + mag_b2 @ comp_w1[69:74] + vel_b2 @ comp_w1[76:84]
    wf = jnp.concatenate([t24, t7, t3, mw, dw, vw, bf], axis=0)
    return wf


def _main_kern(hf_ref, df_ref, dt_ref, v_ref,
               hour_table, circ_w1, circ_b1, circ_w2, circ_b2, day_table,
               wk_w, wk_b, scale_table, mag_w1, mag_b1, mag_w2, mag_b2,
               vel_w1, vel_b1, vel_w2, vel_b2, comp_w1, comp_b1, comp_w2,
               comp_b2, out_ref):
    wf = _fused_weights(hour_table[...], circ_w1[...], circ_b1[...],
                        circ_w2[...], circ_b2[...], day_table[...],
                        wk_w[...], wk_b[...], scale_table[...],
                        mag_w2[...], mag_b2[...], vel_w2[...],
                        vel_b2[...], comp_w1[...], comp_b1[...])
    nt = _NT
    dt = dt_ref[0]
    dc = jnp.clip(dt, 0.0, 24.0)
    mins = dc * 60.0
    sf2 = jnp.where(mins < 5.0, 0, jnp.where(mins < 60.0, 1, 2)).astype(jnp.int32)
    lm2 = jnp.log1p(dc * (1.0 / 24.0))
    m60 = mins - 60.0 * jnp.floor(mins * (1.0 / 60.0))
    a2 = m60 * (2.0 * math.pi / 60.0)
    ac = jnp.concatenate([a2, a2 + 0.5 * math.pi], axis=0)
    scp = jnp.sin(ac)
    r = a2.shape[0]

    hf = hf_ref[0].reshape(1, nt)
    df = df_ref[0].reshape(1, nt)
    sf = sf2.reshape(1, nt)
    v = v_ref[0].reshape(1, nt)
    s = scp[:r].reshape(1, nt)
    c = scp[r:].reshape(1, nt)
    lm = lm2.reshape(1, nt)

    oh24 = (jax.lax.broadcasted_iota(jnp.int32, (24, nt), 0) == hf
            ).astype(jnp.float32)
    oh7 = (jax.lax.broadcasted_iota(jnp.int32, (7, nt), 0) == df
           ).astype(jnp.float32)
    oh3 = (jax.lax.broadcasted_iota(jnp.int32, (3, nt), 0) == sf
           ).astype(jnp.float32)
    hm = jnp.maximum(lm * mag_w1[...].T + mag_b1[...].T, 0.0)
    hv = jnp.maximum(v * vel_w1[...].T + vel_b1[...].T, 0.0)
    ones = jnp.ones((1, nt), jnp.float32)
    ft = jnp.concatenate([oh24, oh7, oh3, hm, s, c, hv, ones], axis=0)

    h1 = jax.lax.dot_general(ft, wf, (((0,), (0,)), ((), ())),
                             preferred_element_type=jnp.float32)
    h1 = jnp.maximum(h1, 0.0)
    out = h1 @ comp_w2[...] + comp_b2[...]
    out_ref[...] = out.reshape(_BB, _SP, 64)[:, :50, :]


def kernel(hours, days, deltas_hours, velocities, hour_table, circ_w1,
           circ_b1, circ_w2, circ_b2, day_table, wk_w, wk_b, scale_table,
           mag_w1, mag_b1, mag_w2, mag_b2, vel_w1, vel_b1, vel_w2, vel_b2,
           comp_w1, comp_b1, comp_w2, comp_b2):
    B, S = hours.shape
    g = B // _BB

    def packed(x):
        xp = jnp.pad(x, ((0, 0), (0, _SP - S)))
        return xp.reshape(g, _BB * _SP // 128, 128)

    def row2(x):
        return x.reshape(1, -1)

    tok_spec = pl.BlockSpec((1, _NT // 128, 128), lambda i: (i, 0, 0))
    full = lambda a: pl.BlockSpec(a.shape, lambda i: tuple(0 for _ in a.shape))
    weights = (hour_table, circ_w1, row2(circ_b1), circ_w2, row2(circ_b2),
               day_table, wk_w, row2(wk_b), scale_table, mag_w1, row2(mag_b1),
               mag_w2, row2(mag_b2), vel_w1, row2(vel_b1), vel_w2,
               row2(vel_b2), comp_w1, row2(comp_b1), comp_w2, row2(comp_b2))
    out = pl.pallas_call(
        _main_kern,
        grid=(g,),
        in_specs=[tok_spec] * 4 + [full(w) for w in weights],
        out_specs=pl.BlockSpec((_BB, S, 64), lambda i: (i, 0, 0)),
        out_shape=jax.ShapeDtypeStruct((B, S, 64), jnp.float32),
        compiler_params=pltpu.CompilerParams(
            dimension_semantics=("arbitrary",)),
    )(packed(hours), packed(days), packed(deltas_hours), packed(velocities),
      *weights)
    return out
